# baseline (speedup 1.0000x reference)
import functools

import numpy as np
import jax
import jax.numpy as jnp
from jax import lax
from jax.experimental import pallas as pl
from jax.experimental.pallas import tpu as pltpu

IRREPS = ((128, 0), (64, 1), (32, 2))
NUM_GRAPHS = 256
EPS = 1e-5


def _round_up(v, m):
    return (v + m - 1) // m * m


def _cdiv(a, b):
    return (a + b - 1) // b


def _structure(irreps):
    D = sum(mul * (2 * l + 1) for mul, l in irreps)
    F = sum(mul for mul, l in irreps)
    rnorm = np.zeros((D, F), np.float32)
    bcast = np.zeros((F, D), np.float32)
    ns = 0
    ix = iw = 0
    for mul, l in irreps:
        d = 2 * l + 1
        for m in range(mul):
            cols = ix + m * d + np.arange(d)
            rnorm[cols, iw + m] = 1.0 / d
            bcast[iw + m, cols] = 1.0
            if l == 0:
                ns += 1
        ix += mul * d
        iw += mul
    return rnorm, bcast, ns, D, F


def _stats_kernel(ns, x_ref, b_ref, sq_ref, cs_ref, cnt_ref):
    @pl.when(pl.program_id(1) == 0)
    def _():
        sq_ref[...] = jnp.zeros_like(sq_ref)
        cs_ref[...] = jnp.zeros_like(cs_ref)
        cnt_ref[...] = jnp.zeros_like(cnt_ref)

    x = x_ref[...].astype(jnp.float32)
    bt = b_ref[...]
    tn = x.shape[0]
    g = sq_ref.shape[0]
    hit = lax.broadcasted_iota(jnp.int32, (tn, g), 1) == bt
    oh = hit.astype(jnp.bfloat16)

    x2 = (x * x).astype(jnp.bfloat16)
    xs = x[:, :ns].astype(jnp.bfloat16)

    dn = (((0,), (0,)), ((), ()))
    sq_ref[...] += lax.dot_general(oh, x2, dn, preferred_element_type=jnp.float32)
    cs_ref[...] += lax.dot_general(oh, xs, dn, preferred_element_type=jnp.float32)
    cnt_ref[...] += jnp.sum(hit.astype(jnp.float32), axis=0, keepdims=True)


def _apply_kernel(ns, x_ref, b_ref, scale_ref, shift_ref, o_ref):
    x = x_ref[...].astype(jnp.float32)
    bt = b_ref[...]
    tn = x.shape[0]
    g = scale_ref.shape[0]
    oh = (lax.broadcasted_iota(jnp.int32, (tn, g), 1) == bt).astype(jnp.bfloat16)

    scale_n = jnp.dot(oh, scale_ref[...], preferred_element_type=jnp.float32)
    shift_n = jnp.dot(oh, shift_ref[...], preferred_element_type=jnp.float32)
    o_ref[:, :ns] = (x[:, :ns] * scale_n[:, :ns] + shift_n).astype(o_ref.dtype)
    o_ref[:, ns:] = (x[:, ns:] * scale_n[:, ns:]).astype(o_ref.dtype)


@functools.partial(jax.jit, static_argnames=('num_graphs', 'irreps', 'eps', 'node_tile'))
def _instance_norm(x, batch, weight, bias, *, num_graphs, irreps, eps, node_tile=4096):
    N, D_in = x.shape
    rnorm_np, bcast_np, ns, D, F = _structure(irreps)
    assert D == D_in

    G = int(num_graphs)
    G_pad = _round_up(max(G, 1), 8)

    n_tiles = _round_up(max(2, _cdiv(N, max(int(node_tile), 8))), 2)
    TN = _round_up(_cdiv(N, n_tiles), 8)
    N_pad = n_tiles * TN
    half = n_tiles // 2

    x_in = x if N_pad == N else jnp.pad(x, ((0, N_pad - N), (0, 0)))
    bt = batch.astype(jnp.int32).reshape(N, 1)
    if N_pad != N:
        bt = jnp.pad(bt, ((0, N_pad - N), (0, 0)), constant_values=G_pad)

    vmem_bytes = 48 * 1024 * 1024
    xbytes = x.dtype.itemsize

    stat_flops = 2 * N_pad * G_pad * (D + ns) + 2 * N_pad * D
    stat_bytes = xbytes * N_pad * D + 4 * N_pad + 4 * 2 * (G_pad * (D + ns) + G_pad)
    part_sq, part_cs, part_cnt = pl.pallas_call(
        functools.partial(_stats_kernel, ns),
        out_shape=(jax.ShapeDtypeStruct((2, G_pad, D), jnp.float32),
                   jax.ShapeDtypeStruct((2, G_pad, ns), jnp.float32),
                   jax.ShapeDtypeStruct((2, 1, G_pad), jnp.float32)),
        grid=(2, half),
        in_specs=[pl.BlockSpec((TN, D), lambda c, i: (c * half + i, 0)),
                  pl.BlockSpec((TN, 1), lambda c, i: (c * half + i, 0))],
        out_specs=(pl.BlockSpec((None, G_pad, D), lambda c, i: (c, 0, 0)),
                   pl.BlockSpec((None, G_pad, ns), lambda c, i: (c, 0, 0)),
                   pl.BlockSpec((None, 1, G_pad), lambda c, i: (c, 0, 0))),
        compiler_params=pltpu.CompilerParams(
            dimension_semantics=("parallel", "arbitrary"),
            vmem_limit_bytes=vmem_bytes),
        cost_estimate=pl.CostEstimate(flops=int(stat_flops), transcendentals=0,
                                      bytes_accessed=int(stat_bytes)),
    )(x_in, bt)

    sq = part_sq.sum(axis=0)
    cs = part_cs.sum(axis=0)
    counts = part_cnt.sum(axis=0).reshape(G_pad, 1)

    rnorm = jnp.asarray(rnorm_np)
    bcast = jnp.asarray(bcast_np)
    w_row = weight.astype(jnp.float32).reshape(1, F)
    b_row = bias.astype(jnp.float32).reshape(1, ns)

    invc = jnp.where(counts > 0, 1.0 / jnp.maximum(counts, 1.0), 0.0)
    mean_s = cs * invc
    msq = sq * invc
    msq = jnp.concatenate([msq[:, :ns] - mean_s * mean_s, msq[:, ns:]], axis=1)
    fnorm = jnp.dot(msq, rnorm)
    scale = lax.rsqrt(fnorm + float(eps)) * w_row
    scale_cols = jnp.dot(scale, bcast)
    shift_s = b_row - mean_s * scale_cols[:, :ns]
    scale_bf = scale_cols.astype(jnp.bfloat16)
    shift_bf = shift_s.astype(jnp.bfloat16)

    apply_flops = 2 * N_pad * G_pad * (D + ns) + 2 * N_pad * D
    apply_bytes = 2 * xbytes * N_pad * D + 4 * N_pad + 2 * G_pad * (D + ns)
    out_pad = pl.pallas_call(
        functools.partial(_apply_kernel, ns),
        out_shape=jax.ShapeDtypeStruct((N_pad, D), x.dtype),
        grid=(n_tiles,),
        in_specs=[pl.BlockSpec((TN, D), lambda i: (i, 0)),
                  pl.BlockSpec((TN, 1), lambda i: (i, 0)),
                  pl.BlockSpec((G_pad, D), lambda i: (0, 0)),
                  pl.BlockSpec((G_pad, ns), lambda i: (0, 0))],
        out_specs=pl.BlockSpec((TN, D), lambda i: (i, 0)),
        compiler_params=pltpu.CompilerParams(
            dimension_semantics=("parallel",),
            vmem_limit_bytes=vmem_bytes),
        cost_estimate=pl.CostEstimate(flops=int(apply_flops), transcendentals=0,
                                      bytes_accessed=int(apply_bytes)),
    )(x_in, bt, scale_bf, shift_bf)

    return out_pad[:N] if N_pad != N else out_pad


def kernel(x, batch, weight, bias):
    return _instance_norm(x, batch, weight, bias, num_graphs=NUM_GRAPHS,
                          irreps=IRREPS, eps=EPS, node_tile=4096)


# speedup vs baseline: 1.0053x; 1.0053x over previous
import functools

import numpy as np
import jax
import jax.numpy as jnp
from jax import lax
from jax.experimental import pallas as pl
from jax.experimental.pallas import tpu as pltpu

IRREPS = ((128, 0), (64, 1), (32, 2))
NUM_GRAPHS = 256
EPS = 1e-5


def _round_up(v, m):
    return (v + m - 1) // m * m


def _cdiv(a, b):
    return (a + b - 1) // b


def _structure(irreps):
    D = sum(mul * (2 * l + 1) for mul, l in irreps)
    F = sum(mul for mul, l in irreps)
    rnorm = np.zeros((D, F), np.float32)
    bcast = np.zeros((F, D), np.float32)
    ns = 0
    ix = iw = 0
    for mul, l in irreps:
        d = 2 * l + 1
        for m in range(mul):
            cols = ix + m * d + np.arange(d)
            rnorm[cols, iw + m] = 1.0 / d
            bcast[iw + m, cols] = 1.0
            if l == 0:
                ns += 1
        ix += mul * d
        iw += mul
    return rnorm, bcast, ns, D, F


def _stats_kernel(ns, x_ref, b_ref, sq_ref, cs_ref, cnt_ref):
    @pl.when(pl.program_id(1) == 0)
    def _():
        sq_ref[...] = jnp.zeros_like(sq_ref)
        cs_ref[...] = jnp.zeros_like(cs_ref)
        cnt_ref[...] = jnp.zeros_like(cnt_ref)

    x = x_ref[...].astype(jnp.float32)
    bt = b_ref[...]
    tn = x.shape[0]
    g = sq_ref.shape[0]
    hit = lax.broadcasted_iota(jnp.int32, (tn, g), 1) == bt
    oh = hit.astype(jnp.bfloat16)

    x2 = (x * x).astype(jnp.bfloat16)
    xs = x[:, :ns].astype(jnp.bfloat16)

    dn = (((0,), (0,)), ((), ()))
    sq_ref[...] += lax.dot_general(oh, x2, dn, preferred_element_type=jnp.float32)
    cs_ref[...] += lax.dot_general(oh, xs, dn, preferred_element_type=jnp.float32)
    cnt_ref[...] += jnp.sum(hit.astype(jnp.float32), axis=0, keepdims=True)


def _apply_kernel(ns, eps, x_ref, b_ref, sq_ref, cs_ref, cnt_ref, w_ref,
                  bias_ref, rnorm_ref, bcast_ref, o_ref, scale_scr, shift_scr):
    @pl.when(pl.program_id(1) == 0)
    def _():
        sq = sq_ref[0] + sq_ref[1]
        cs = cs_ref[0] + cs_ref[1]
        cnt_row = cnt_ref[0] + cnt_ref[1]
        cnt = cnt_row.T
        invc = jnp.where(cnt > 0, 1.0 / jnp.maximum(cnt, 1.0), 0.0)
        mean_s = cs * invc
        msq = jnp.concatenate(
            [sq[:, :ns] * invc - mean_s * mean_s, sq[:, ns:] * invc], axis=1)
        fnorm = jnp.dot(msq, rnorm_ref[...], preferred_element_type=jnp.float32)
        scale = lax.rsqrt(fnorm + eps) * w_ref[...]
        scale_cols = jnp.dot(scale, bcast_ref[...],
                             preferred_element_type=jnp.float32)
        shift = bias_ref[...] - mean_s * scale_cols[:, :ns]
        scale_scr[...] = scale_cols.astype(jnp.bfloat16)
        shift_scr[...] = shift.astype(jnp.bfloat16)

    x = x_ref[...].astype(jnp.float32)
    bt = b_ref[...]
    tn = x.shape[0]
    g = scale_scr.shape[0]
    oh = (lax.broadcasted_iota(jnp.int32, (tn, g), 1) == bt).astype(jnp.bfloat16)

    scale_n = jnp.dot(oh, scale_scr[...], preferred_element_type=jnp.float32)
    shift_n = jnp.dot(oh, shift_scr[...], preferred_element_type=jnp.float32)
    o_ref[:, :ns] = (x[:, :ns] * scale_n[:, :ns] + shift_n).astype(o_ref.dtype)
    o_ref[:, ns:] = (x[:, ns:] * scale_n[:, ns:]).astype(o_ref.dtype)


@functools.partial(jax.jit, static_argnames=('num_graphs', 'irreps', 'eps', 'node_tile'))
def _instance_norm(x, batch, weight, bias, *, num_graphs, irreps, eps, node_tile=4096):
    N, D_in = x.shape
    rnorm_np, bcast_np, ns, D, F = _structure(irreps)
    assert D == D_in

    G = int(num_graphs)
    G_pad = _round_up(max(G, 1), 8)

    n_tiles = _round_up(max(2, _cdiv(N, max(int(node_tile), 8))), 2)
    TN = _round_up(_cdiv(N, n_tiles), 8)
    N_pad = n_tiles * TN
    half = n_tiles // 2

    x_in = x if N_pad == N else jnp.pad(x, ((0, N_pad - N), (0, 0)))
    bt = batch.astype(jnp.int32).reshape(N, 1)
    if N_pad != N:
        bt = jnp.pad(bt, ((0, N_pad - N), (0, 0)), constant_values=G_pad)

    vmem_bytes = 48 * 1024 * 1024
    xbytes = x.dtype.itemsize

    stat_flops = 2 * N_pad * G_pad * (D + ns) + 2 * N_pad * D
    stat_bytes = xbytes * N_pad * D + 4 * N_pad + 4 * 2 * (G_pad * (D + ns) + G_pad)
    part_sq, part_cs, part_cnt = pl.pallas_call(
        functools.partial(_stats_kernel, ns),
        out_shape=(jax.ShapeDtypeStruct((2, G_pad, D), jnp.float32),
                   jax.ShapeDtypeStruct((2, G_pad, ns), jnp.float32),
                   jax.ShapeDtypeStruct((2, 1, G_pad), jnp.float32)),
        grid=(2, half),
        in_specs=[pl.BlockSpec((TN, D), lambda c, i: (c * half + i, 0)),
                  pl.BlockSpec((TN, 1), lambda c, i: (c * half + i, 0))],
        out_specs=(pl.BlockSpec((None, G_pad, D), lambda c, i: (c, 0, 0)),
                   pl.BlockSpec((None, G_pad, ns), lambda c, i: (c, 0, 0)),
                   pl.BlockSpec((None, 1, G_pad), lambda c, i: (c, 0, 0))),
        compiler_params=pltpu.CompilerParams(
            dimension_semantics=("parallel", "arbitrary"),
            vmem_limit_bytes=vmem_bytes),
        cost_estimate=pl.CostEstimate(flops=int(stat_flops), transcendentals=0,
                                      bytes_accessed=int(stat_bytes)),
    )(x_in, bt)

    w_row = weight.astype(jnp.float32).reshape(1, F)
    b_row = bias.astype(jnp.float32).reshape(1, ns)
    rnorm = jnp.asarray(rnorm_np)
    bcast = jnp.asarray(bcast_np)

    apply_flops = 2 * N_pad * G_pad * (D + ns) + 2 * N_pad * D
    apply_bytes = 2 * xbytes * N_pad * D + 4 * N_pad + 4 * 2 * G_pad * (D + ns)
    out_pad = pl.pallas_call(
        functools.partial(_apply_kernel, ns, float(eps)),
        out_shape=jax.ShapeDtypeStruct((N_pad, D), x.dtype),
        grid=(2, half),
        in_specs=[pl.BlockSpec((TN, D), lambda c, i: (c * half + i, 0)),
                  pl.BlockSpec((TN, 1), lambda c, i: (c * half + i, 0)),
                  pl.BlockSpec((2, G_pad, D), lambda c, i: (0, 0, 0)),
                  pl.BlockSpec((2, G_pad, ns), lambda c, i: (0, 0, 0)),
                  pl.BlockSpec((2, 1, G_pad), lambda c, i: (0, 0, 0)),
                  pl.BlockSpec((1, F), lambda c, i: (0, 0)),
                  pl.BlockSpec((1, ns), lambda c, i: (0, 0)),
                  pl.BlockSpec((D, F), lambda c, i: (0, 0)),
                  pl.BlockSpec((F, D), lambda c, i: (0, 0))],
        out_specs=pl.BlockSpec((TN, D), lambda c, i: (c * half + i, 0)),
        scratch_shapes=[pltpu.VMEM((G_pad, D), jnp.bfloat16),
                        pltpu.VMEM((G_pad, ns), jnp.bfloat16)],
        compiler_params=pltpu.CompilerParams(
            dimension_semantics=("parallel", "arbitrary"),
            vmem_limit_bytes=vmem_bytes),
        cost_estimate=pl.CostEstimate(flops=int(apply_flops), transcendentals=0,
                                      bytes_accessed=int(apply_bytes)),
    )(x_in, bt, part_sq, part_cs, part_cnt, w_row, b_row, rnorm, bcast)

    return out_pad[:N] if N_pad != N else out_pad


def kernel(x, batch, weight, bias):
    return _instance_norm(x, batch, weight, bias, num_graphs=NUM_GRAPHS,
                          irreps=IRREPS, eps=EPS, node_tile=4096)


# speedup vs baseline: 1.0742x; 1.0685x over previous
import functools

import numpy as np
import jax
import jax.numpy as jnp
from jax import lax
from jax.experimental import pallas as pl
from jax.experimental.pallas import tpu as pltpu

IRREPS = ((128, 0), (64, 1), (32, 2))
NUM_GRAPHS = 256
EPS = 1e-5


def _round_up(v, m):
    return (v + m - 1) // m * m


def _cdiv(a, b):
    return (a + b - 1) // b


def _structure(irreps):
    D = sum(mul * (2 * l + 1) for mul, l in irreps)
    F = sum(mul for mul, l in irreps)
    rnorm = np.zeros((D, F), np.float32)
    bcast = np.zeros((F, D), np.float32)
    ns = 0
    ix = iw = 0
    for mul, l in irreps:
        d = 2 * l + 1
        for m in range(mul):
            cols = ix + m * d + np.arange(d)
            rnorm[cols, iw + m] = 1.0 / d
            bcast[iw + m, cols] = 1.0
            if l == 0:
                ns += 1
        ix += mul * d
        iw += mul
    return rnorm, bcast, ns, D, F


def _fused_kernel(ns, eps, T, TN,
                  x_ref, b_ref, w_ref, bias_ref, rnorm_ref, bcast_ref,
                  o_ref,
                  xbf_scr, sq_scr, cs_scr, cnt_scr, scale_scr, shift_scr):
    i = pl.program_id(0)
    g = sq_scr.shape[0]
    dn = (((0,), (0,)), ((), ()))

    @pl.when(i == 0)
    def _init():
        sq_scr[...] = jnp.zeros_like(sq_scr)
        cs_scr[...] = jnp.zeros_like(cs_scr)
        cnt_scr[...] = jnp.zeros_like(cnt_scr)

    @pl.when(i < T)
    def _stats():
        x = x_ref[...].astype(jnp.float32)
        off = pl.multiple_of(i * TN, TN)
        xbf_scr[pl.ds(off, TN), :] = x.astype(jnp.bfloat16)
        bt = b_ref[...]
        hit = lax.broadcasted_iota(jnp.int32, (TN, g), 1) == bt
        oh = hit.astype(jnp.bfloat16)
        x2 = (x * x).astype(jnp.bfloat16)
        xs = x[:, :ns].astype(jnp.bfloat16)
        sq_scr[...] += lax.dot_general(oh, x2, dn, preferred_element_type=jnp.float32)
        cs_scr[...] += lax.dot_general(oh, xs, dn, preferred_element_type=jnp.float32)
        cnt_scr[...] += jnp.sum(hit.astype(jnp.float32), axis=0, keepdims=True)

    @pl.when(i == T)
    def _mid():
        sq = sq_scr[...]
        cs = cs_scr[...]
        cnt = cnt_scr[...].T
        invc = jnp.where(cnt > 0, 1.0 / jnp.maximum(cnt, 1.0), 0.0)
        mean_s = cs * invc
        msq = jnp.concatenate(
            [sq[:, :ns] * invc - mean_s * mean_s, sq[:, ns:] * invc], axis=1)
        fnorm = jnp.dot(msq, rnorm_ref[...], preferred_element_type=jnp.float32)
        scale = lax.rsqrt(fnorm + eps) * w_ref[...]
        scale_cols = jnp.dot(scale, bcast_ref[...],
                             preferred_element_type=jnp.float32)
        shift = bias_ref[...] - mean_s * scale_cols[:, :ns]
        scale_scr[...] = scale_cols.astype(jnp.bfloat16)
        shift_scr[...] = shift.astype(jnp.bfloat16)

    @pl.when(i >= T)
    def _apply():
        t = i - T
        off = pl.multiple_of(t * TN, TN)
        xb = xbf_scr[pl.ds(off, TN), :].astype(jnp.float32)
        bt = b_ref[...]
        oh = (lax.broadcasted_iota(jnp.int32, (TN, g), 1) == bt).astype(jnp.bfloat16)
        scale_n = jnp.dot(oh, scale_scr[...], preferred_element_type=jnp.float32)
        shift_n = jnp.dot(oh, shift_scr[...], preferred_element_type=jnp.float32)
        o_ref[:, :ns] = (xb[:, :ns] * scale_n[:, :ns] + shift_n).astype(o_ref.dtype)
        o_ref[:, ns:] = (xb[:, ns:] * scale_n[:, ns:]).astype(o_ref.dtype)


@functools.partial(jax.jit, static_argnames=('num_graphs', 'irreps', 'eps', 'node_tile'))
def _instance_norm(x, batch, weight, bias, *, num_graphs, irreps, eps, node_tile=2048):
    N, D_in = x.shape
    rnorm_np, bcast_np, ns, D, F = _structure(irreps)
    assert D == D_in

    G = int(num_graphs)
    G_pad = _round_up(max(G, 1), 8)

    T = max(1, _cdiv(N, max(int(node_tile), 8)))
    TN = _round_up(_cdiv(N, T), 8)
    N_pad = T * TN

    x_in = x if N_pad == N else jnp.pad(x, ((0, N_pad - N), (0, 0)))
    bt = batch.astype(jnp.int32).reshape(N, 1)
    if N_pad != N:
        bt = jnp.pad(bt, ((0, N_pad - N), (0, 0)), constant_values=G_pad)

    w_row = weight.astype(jnp.float32).reshape(1, F)
    b_row = bias.astype(jnp.float32).reshape(1, ns)
    rnorm = jnp.asarray(rnorm_np)
    bcast = jnp.asarray(bcast_np)

    flops = 4 * N_pad * G_pad * (D + ns) + 4 * N_pad * D
    bytes_accessed = 2 * x.dtype.itemsize * N_pad * D + 8 * N_pad
    Tm1 = T - 1

    out_pad = pl.pallas_call(
        functools.partial(_fused_kernel, ns, float(eps), T, TN),
        out_shape=jax.ShapeDtypeStruct((N_pad, D), x.dtype),
        grid=(2 * T,),
        in_specs=[
            pl.BlockSpec((TN, D), lambda i: (jnp.minimum(i, Tm1), 0)),
            pl.BlockSpec((TN, 1), lambda i: (jnp.where(i < T, i, i - T), 0)),
            pl.BlockSpec((1, F), lambda i: (0, 0)),
            pl.BlockSpec((1, ns), lambda i: (0, 0)),
            pl.BlockSpec((D, F), lambda i: (0, 0)),
            pl.BlockSpec((F, D), lambda i: (0, 0)),
        ],
        out_specs=pl.BlockSpec((TN, D), lambda i: (jnp.where(i < T, 0, i - T), 0)),
        scratch_shapes=[
            pltpu.VMEM((N_pad, D), jnp.bfloat16),
            pltpu.VMEM((G_pad, D), jnp.float32),
            pltpu.VMEM((G_pad, ns), jnp.float32),
            pltpu.VMEM((1, G_pad), jnp.float32),
            pltpu.VMEM((G_pad, D), jnp.bfloat16),
            pltpu.VMEM((G_pad, ns), jnp.bfloat16),
        ],
        compiler_params=pltpu.CompilerParams(
            dimension_semantics=("arbitrary",),
            vmem_limit_bytes=58 * 1024 * 1024),
        cost_estimate=pl.CostEstimate(flops=int(flops), transcendentals=0,
                                      bytes_accessed=int(bytes_accessed)),
    )(x_in, bt, w_row, b_row, rnorm, bcast)

    return out_pad[:N] if N_pad != N else out_pad


def kernel(x, batch, weight, bias):
    return _instance_norm(x, batch, weight, bias, num_graphs=NUM_GRAPHS,
                          irreps=IRREPS, eps=EPS, node_tile=2048)


# speedup vs baseline: 1.0760x; 1.0017x over previous
import functools

import numpy as np
import jax
import jax.numpy as jnp
from jax import lax
from jax.experimental import pallas as pl
from jax.experimental.pallas import tpu as pltpu

IRREPS = ((128, 0), (64, 1), (32, 2))
NUM_GRAPHS = 256
EPS = 1e-5


def _round_up(v, m):
    return (v + m - 1) // m * m


def _cdiv(a, b):
    return (a + b - 1) // b


def _structure(irreps):
    D = sum(mul * (2 * l + 1) for mul, l in irreps)
    F = sum(mul for mul, l in irreps)
    rnorm = np.zeros((D, F), np.float32)
    bcast = np.zeros((F, D), np.float32)
    ns = 0
    ix = iw = 0
    for mul, l in irreps:
        d = 2 * l + 1
        for m in range(mul):
            cols = ix + m * d + np.arange(d)
            rnorm[cols, iw + m] = 1.0 / d
            bcast[iw + m, cols] = 1.0
            if l == 0:
                ns += 1
        ix += mul * d
        iw += mul
    return rnorm, bcast, ns, D, F


def _fused_kernel(ns, eps, T, TN,
                  x_ref, b_ref, w_ref, bias_ref, rnorm_ref, bcast_ref,
                  o_ref,
                  xbf_scr, acc_scr, cnt_scr, tab_scr):
    i = pl.program_id(0)
    g = acc_scr.shape[0]
    dn = (((0,), (0,)), ((), ()))

    @pl.when(i == 0)
    def _init():
        acc_scr[...] = jnp.zeros_like(acc_scr)
        cnt_scr[...] = jnp.zeros_like(cnt_scr)

    @pl.when(i < T)
    def _stats():
        x = x_ref[...].astype(jnp.float32)
        off = pl.multiple_of(i * TN, TN)
        xbf_scr[pl.ds(off, TN), :] = x.astype(jnp.bfloat16)
        bt = b_ref[...]
        hit = lax.broadcasted_iota(jnp.int32, (TN, g), 1) == bt
        oh = hit.astype(jnp.bfloat16)
        y = jnp.concatenate(
            [x[:, :ns].astype(jnp.bfloat16),
             (x * x).astype(jnp.bfloat16)], axis=1)
        acc_scr[...] += lax.dot_general(oh, y, dn, preferred_element_type=jnp.float32)
        cnt_scr[...] += jnp.sum(hit, axis=0, keepdims=True)

    @pl.when(i == T)
    def _mid():
        cs = acc_scr[:, :ns]
        sq = acc_scr[:, ns:]
        cnt = cnt_scr[...].astype(jnp.float32).T
        invc = jnp.where(cnt > 0, 1.0 / jnp.maximum(cnt, 1.0), 0.0)
        mean_s = cs * invc
        msq = jnp.concatenate(
            [sq[:, :ns] * invc - mean_s * mean_s, sq[:, ns:] * invc], axis=1)
        fnorm = jnp.dot(msq, rnorm_ref[...], preferred_element_type=jnp.float32)
        scale = lax.rsqrt(fnorm + eps) * w_ref[...]
        scale_cols = jnp.dot(scale, bcast_ref[...],
                             preferred_element_type=jnp.float32)
        shift = bias_ref[...] - mean_s * scale_cols[:, :ns]
        tab_scr[:, :ns] = shift.astype(jnp.bfloat16)
        tab_scr[:, ns:] = scale_cols.astype(jnp.bfloat16)

    @pl.when(i >= T)
    def _apply():
        t = i - T
        off = pl.multiple_of(t * TN, TN)
        xb = xbf_scr[pl.ds(off, TN), :].astype(jnp.float32)
        bt = b_ref[...]
        oh = (lax.broadcasted_iota(jnp.int32, (TN, g), 1) == bt).astype(jnp.bfloat16)
        res = jnp.dot(oh, tab_scr[...], preferred_element_type=jnp.float32)
        shift_n = res[:, :ns]
        scale_n = res[:, ns:]
        o_ref[:, :ns] = (xb[:, :ns] * scale_n[:, :ns] + shift_n).astype(o_ref.dtype)
        o_ref[:, ns:] = (xb[:, ns:] * scale_n[:, ns:]).astype(o_ref.dtype)


@functools.partial(jax.jit, static_argnames=('num_graphs', 'irreps', 'eps', 'node_tile'))
def _instance_norm(x, batch, weight, bias, *, num_graphs, irreps, eps, node_tile=2048):
    N, D_in = x.shape
    rnorm_np, bcast_np, ns, D, F = _structure(irreps)
    assert D == D_in

    G = int(num_graphs)
    G_pad = _round_up(max(G, 1), 8)

    T = max(1, _cdiv(N, max(int(node_tile), 8)))
    TN = _round_up(_cdiv(N, T), 8)
    N_pad = T * TN

    x_in = x if N_pad == N else jnp.pad(x, ((0, N_pad - N), (0, 0)))
    bt = batch.astype(jnp.int32).reshape(N, 1)
    if N_pad != N:
        bt = jnp.pad(bt, ((0, N_pad - N), (0, 0)), constant_values=G_pad)

    w_row = weight.astype(jnp.float32).reshape(1, F)
    b_row = bias.astype(jnp.float32).reshape(1, ns)
    rnorm = jnp.asarray(rnorm_np)
    bcast = jnp.asarray(bcast_np)

    flops = 4 * N_pad * G_pad * (D + ns) + 4 * N_pad * D
    bytes_accessed = 2 * x.dtype.itemsize * N_pad * D + 8 * N_pad
    Tm1 = T - 1

    out_pad = pl.pallas_call(
        functools.partial(_fused_kernel, ns, float(eps), T, TN),
        out_shape=jax.ShapeDtypeStruct((N_pad, D), x.dtype),
        grid=(2 * T,),
        in_specs=[
            pl.BlockSpec((TN, D), lambda i: (jnp.minimum(i, Tm1), 0)),
            pl.BlockSpec((TN, 1), lambda i: (jnp.where(i < T, i, i - T), 0)),
            pl.BlockSpec((1, F), lambda i: (0, 0)),
            pl.BlockSpec((1, ns), lambda i: (0, 0)),
            pl.BlockSpec((D, F), lambda i: (0, 0)),
            pl.BlockSpec((F, D), lambda i: (0, 0)),
        ],
        out_specs=pl.BlockSpec((TN, D), lambda i: (jnp.where(i < T, 0, i - T), 0)),
        scratch_shapes=[
            pltpu.VMEM((N_pad, D), jnp.bfloat16),
            pltpu.VMEM((G_pad, ns + D), jnp.float32),
            pltpu.VMEM((1, G_pad), jnp.int32),
            pltpu.VMEM((G_pad, ns + D), jnp.bfloat16),
        ],
        compiler_params=pltpu.CompilerParams(
            dimension_semantics=("arbitrary",),
            vmem_limit_bytes=58 * 1024 * 1024),
        cost_estimate=pl.CostEstimate(flops=int(flops), transcendentals=0,
                                      bytes_accessed=int(bytes_accessed)),
    )(x_in, bt, w_row, b_row, rnorm, bcast)

    return out_pad[:N] if N_pad != N else out_pad


def kernel(x, batch, weight, bias):
    return _instance_norm(x, batch, weight, bias, num_graphs=NUM_GRAPHS,
                          irreps=IRREPS, eps=EPS, node_tile=2048)


# speedup vs baseline: 1.2205x; 1.1343x over previous
import functools

import numpy as np
import jax
import jax.numpy as jnp
from jax import lax
from jax.experimental import pallas as pl
from jax.experimental.pallas import tpu as pltpu

IRREPS = ((128, 0), (64, 1), (32, 2))
NUM_GRAPHS = 256
EPS = 1e-5


def _round_up(v, m):
    return (v + m - 1) // m * m


def _cdiv(a, b):
    return (a + b - 1) // b


def _structure(irreps):
    D = sum(mul * (2 * l + 1) for mul, l in irreps)
    F = sum(mul for mul, l in irreps)
    rnorm = np.zeros((D, F), np.float32)
    bcast = np.zeros((F, D), np.float32)
    ns = 0
    ix = iw = 0
    for mul, l in irreps:
        d = 2 * l + 1
        for m in range(mul):
            cols = ix + m * d + np.arange(d)
            rnorm[cols, iw + m] = 1.0 / d
            bcast[iw + m, cols] = 1.0
            if l == 0:
                ns += 1
        ix += mul * d
        iw += mul
    return rnorm, bcast, ns, D, F


def _fused_kernel(ns, eps, T, TN,
                  x_ref, b_ref, w_ref, bias_ref, rnorm_ref, bcast_ref,
                  o_ref,
                  xbf_scr, acc_scr, cnt_scr, tab_scr):
    i = pl.program_id(0)
    g = acc_scr.shape[0]

    @pl.when(i == 0)
    def _init():
        acc_scr[...] = jnp.zeros_like(acc_scr)
        cnt_scr[...] = jnp.zeros_like(cnt_scr)

    @pl.when(i < T)
    def _stats():
        x = x_ref[...].astype(jnp.float32)
        off = pl.multiple_of(i * TN, TN)
        xbf_scr[pl.ds(off, TN), :] = x.astype(jnp.bfloat16)
        btr = b_ref[...]
        hit = lax.broadcasted_iota(jnp.int32, (g, TN), 0) == btr
        oht = hit.astype(jnp.bfloat16)
        y = jnp.concatenate(
            [x[:, :ns].astype(jnp.bfloat16),
             (x * x).astype(jnp.bfloat16)], axis=1)
        acc_scr[...] += jnp.dot(oht, y, preferred_element_type=jnp.float32)
        cnt_scr[...] += jnp.sum(hit, axis=1, keepdims=True)

    @pl.when(i == T)
    def _mid():
        cs = acc_scr[:, :ns]
        sq = acc_scr[:, ns:]
        cnt = cnt_scr[...].astype(jnp.float32)
        invc = jnp.where(cnt > 0, 1.0 / jnp.maximum(cnt, 1.0), 0.0)
        mean_s = cs * invc
        msq = jnp.concatenate(
            [sq[:, :ns] * invc - mean_s * mean_s, sq[:, ns:] * invc], axis=1)
        fnorm = jnp.dot(msq, rnorm_ref[...], preferred_element_type=jnp.float32)
        scale = lax.rsqrt(fnorm + eps) * w_ref[...]
        scale_cols = jnp.dot(scale, bcast_ref[...],
                             preferred_element_type=jnp.float32)
        shift = bias_ref[...] - mean_s * scale_cols[:, :ns]
        tab_scr[:, :ns] = shift.astype(jnp.bfloat16)
        tab_scr[:, ns:] = scale_cols.astype(jnp.bfloat16)

    @pl.when(i >= T)
    def _apply():
        t = i - T
        off = pl.multiple_of(t * TN, TN)
        xb = xbf_scr[pl.ds(off, TN), :].astype(jnp.float32)
        btr = b_ref[...]
        oht = (lax.broadcasted_iota(jnp.int32, (g, TN), 0) == btr).astype(jnp.bfloat16)
        dn = (((0,), (0,)), ((), ()))
        res = lax.dot_general(oht, tab_scr[...], dn,
                              preferred_element_type=jnp.float32)
        shift_n = res[:, :ns]
        scale_n = res[:, ns:]
        o_ref[:, :ns] = (xb[:, :ns] * scale_n[:, :ns] + shift_n).astype(o_ref.dtype)
        o_ref[:, ns:] = (xb[:, ns:] * scale_n[:, ns:]).astype(o_ref.dtype)


@functools.partial(jax.jit, static_argnames=('num_graphs', 'irreps', 'eps', 'node_tile'))
def _instance_norm(x, batch, weight, bias, *, num_graphs, irreps, eps, node_tile=2048):
    N, D_in = x.shape
    rnorm_np, bcast_np, ns, D, F = _structure(irreps)
    assert D == D_in

    G = int(num_graphs)
    G_pad = _round_up(max(G, 1), 8)

    T = max(1, _cdiv(N, max(int(node_tile), 8)))
    TN = _round_up(_cdiv(N, T), 8)
    N_pad = T * TN

    x_in = x if N_pad == N else jnp.pad(x, ((0, N_pad - N), (0, 0)))
    bt = batch.astype(jnp.int32)
    if N_pad != N:
        bt = jnp.pad(bt, (0, N_pad - N), constant_values=G_pad)
    btT = bt.reshape(T, 1, TN)

    w_row = weight.astype(jnp.float32).reshape(1, F)
    b_row = bias.astype(jnp.float32).reshape(1, ns)
    rnorm = jnp.asarray(rnorm_np)
    bcast = jnp.asarray(bcast_np)

    flops = 4 * N_pad * G_pad * (D + ns) + 4 * N_pad * D
    bytes_accessed = 2 * x.dtype.itemsize * N_pad * D + 8 * N_pad
    Tm1 = T - 1

    out_pad = pl.pallas_call(
        functools.partial(_fused_kernel, ns, float(eps), T, TN),
        out_shape=jax.ShapeDtypeStruct((N_pad, D), x.dtype),
        grid=(2 * T,),
        in_specs=[
            pl.BlockSpec((TN, D), lambda i: (jnp.minimum(i, Tm1), 0)),
            pl.BlockSpec((None, 1, TN), lambda i: (jnp.where(i < T, i, i - T), 0, 0)),
            pl.BlockSpec((1, F), lambda i: (0, 0)),
            pl.BlockSpec((1, ns), lambda i: (0, 0)),
            pl.BlockSpec((D, F), lambda i: (0, 0)),
            pl.BlockSpec((F, D), lambda i: (0, 0)),
        ],
        out_specs=pl.BlockSpec((TN, D), lambda i: (jnp.where(i < T, 0, i - T), 0)),
        scratch_shapes=[
            pltpu.VMEM((N_pad, D), jnp.bfloat16),
            pltpu.VMEM((G_pad, ns + D), jnp.float32),
            pltpu.VMEM((G_pad, 1), jnp.int32),
            pltpu.VMEM((G_pad, ns + D), jnp.bfloat16),
        ],
        compiler_params=pltpu.CompilerParams(
            dimension_semantics=("arbitrary",),
            vmem_limit_bytes=58 * 1024 * 1024),
        cost_estimate=pl.CostEstimate(flops=int(flops), transcendentals=0,
                                      bytes_accessed=int(bytes_accessed)),
    )(x_in, btT, w_row, b_row, rnorm, bcast)

    return out_pad[:N] if N_pad != N else out_pad


def kernel(x, batch, weight, bias):
    return _instance_norm(x, batch, weight, bias, num_graphs=NUM_GRAPHS,
                          irreps=IRREPS, eps=EPS, node_tile=2048)
